# baseline (speedup 1.0000x reference)
import functools

import jax
import jax.numpy as jnp
from jax import lax
from jax.experimental import pallas as pl
from jax.experimental.pallas import tpu as pltpu
from jax.experimental.pallas import tpu_sc as plsc

EMB = 768
NE = 8
TOPK = 2
FF = 768
S = 2048
EP = 128

TS = 256
NRT = S // TS

TM = 256
NT = (TOPK * S) // TM + NE
P = NT * TM

NW = 32
TOKW = S // NW


def _router_body(x_ref, rwt_ref, rb_ref, meta_ref, counts_ref, q0r_ref,
                 q1r_ref):
    x = x_ref[...]
    scores = jnp.dot(x, rwt_ref[...], preferred_element_type=jnp.float32)
    lane = lax.broadcasted_iota(jnp.int32, (TS, EP), 1)
    valid = lane < NE
    scores = scores + rb_ref[...]
    scores = jnp.where(valid, scores, -1e30)
    m = jnp.max(scores, axis=1, keepdims=True)
    ex = jnp.where(valid, jnp.exp(scores - m), 0.0)
    probs = ex / jnp.sum(ex, axis=1, keepdims=True)

    m1 = jnp.max(probs, axis=1, keepdims=True)
    a1 = jnp.min(jnp.where(probs == m1, lane, EP), axis=1, keepdims=True)
    oh1 = (lane == a1).astype(jnp.float32)
    probs2 = jnp.where(lane == a1, -1.0, probs)
    m2 = jnp.max(probs2, axis=1, keepdims=True)
    a2 = jnp.min(jnp.where(probs2 == m2, lane, EP), axis=1, keepdims=True)
    oh2 = (lane == a2).astype(jnp.float32)
    oh = oh1 + oh2

    counts_ref[...] = jnp.sum(oh, axis=0, keepdims=True)[None]

    r = lax.broadcasted_iota(jnp.int32, (TS, TS), 0)
    c = lax.broadcasted_iota(jnp.int32, (TS, TS), 1)
    tri = (c < r).astype(jnp.float32)
    cum = jnp.dot(tri, oh, preferred_element_type=jnp.float32)
    r1 = jnp.sum(cum * oh1, axis=1, keepdims=True)
    r2 = jnp.sum(cum * oh2, axis=1, keepdims=True)

    meta_ref[...] = (
        jnp.where(lane == 0, m1, 0.0)
        + jnp.where(lane == 1, m2, 0.0)
        + jnp.where(lane == 2, a1.astype(jnp.float32), 0.0)
        + jnp.where(lane == 3, a2.astype(jnp.float32), 0.0)
        + jnp.where(lane == 4, r1, 0.0)
        + jnp.where(lane == 5, r2, 0.0)
    )
    q0r_ref[...] = jnp.broadcast_to(m1, (TS, EP))
    q1r_ref[...] = jnp.broadcast_to(m2, (TS, EP))


def _router(xf, rwt, rbp):
    return pl.pallas_call(
        _router_body,
        grid=(NRT,),
        in_specs=[
            pl.BlockSpec((TS, EMB), lambda g: (g, 0)),
            pl.BlockSpec((EMB, EP), lambda g: (0, 0)),
            pl.BlockSpec((1, EP), lambda g: (0, 0)),
        ],
        out_specs=[
            pl.BlockSpec((TS, EP), lambda g: (g, 0)),
            pl.BlockSpec((1, 1, EP), lambda g: (g, 0, 0)),
            pl.BlockSpec((TS, EP), lambda g: (g, 0)),
            pl.BlockSpec((TS, EP), lambda g: (g, 0)),
        ],
        out_shape=[
            jax.ShapeDtypeStruct((S, EP), jnp.float32),
            jax.ShapeDtypeStruct((NRT, 1, EP), jnp.float32),
            jax.ShapeDtypeStruct((S, EP), jnp.float32),
            jax.ShapeDtypeStruct((S, EP), jnp.float32),
        ],
    )(xf, rwt, rbp)


def _dispatch_body(x_hbm, pos0_hbm, pos1_hbm, q0r_hbm, q1r_hbm, xs_hbm, sp_hbm,
                   rows_v, p0_v, p1_v, q0_v, q1_v, sem0, sem1, sem2, sem3):
    wid = lax.axis_index("s") * 2 + lax.axis_index("c")
    base = wid * TOKW
    pltpu.sync_copy(x_hbm.at[pl.ds(base, TOKW)], rows_v)
    pltpu.sync_copy(pos0_hbm.at[pl.ds(base, TOKW)], p0_v)
    pltpu.sync_copy(pos1_hbm.at[pl.ds(base, TOKW)], p1_v)
    pltpu.sync_copy(q0r_hbm.at[pl.ds(base, TOKW)], q0_v)
    pltpu.sync_copy(q1r_hbm.at[pl.ds(base, TOKW)], q1_v)
    c0 = pltpu.async_copy(rows_v, xs_hbm.at[p0_v], sem0)
    c1 = pltpu.async_copy(rows_v, xs_hbm.at[p1_v], sem1)
    c2 = pltpu.async_copy(q0_v, sp_hbm.at[p0_v], sem2)
    c3 = pltpu.async_copy(q1_v, sp_hbm.at[p1_v], sem3)
    c0.wait()
    c1.wait()
    c2.wait()
    c3.wait()


def _dispatch(xf, pos0, pos1, q0r, q1r):
    mesh = plsc.VectorSubcoreMesh(core_axis_name="c", subcore_axis_name="s")
    fn = functools.partial(
        pl.kernel,
        mesh=mesh,
        out_type=[
            jax.ShapeDtypeStruct((P, EMB), jnp.float32),
            jax.ShapeDtypeStruct((P, 128), jnp.float32),
        ],
        scratch_types=[
            pltpu.VMEM((TOKW, EMB), jnp.float32),
            pltpu.VMEM((TOKW,), jnp.int32),
            pltpu.VMEM((TOKW,), jnp.int32),
            pltpu.VMEM((TOKW, 128), jnp.float32),
            pltpu.VMEM((TOKW, 128), jnp.float32),
            pltpu.SemaphoreType.DMA,
            pltpu.SemaphoreType.DMA,
            pltpu.SemaphoreType.DMA,
            pltpu.SemaphoreType.DMA,
        ],
    )(_dispatch_body)
    return fn(xf, pos0, pos1, q0r, q1r)


def _ffn_body(te_ref, xs_ref, sp_ref, w1_ref, b1_ref, ws_ref, bs_ref, w2_ref,
              b2_ref, ys_ref):
    xv = xs_ref[...]
    h1 = lax.dot_general(xv, w1_ref[0], (((1,), (1,)), ((), ())),
                         preferred_element_type=jnp.float32,
                         precision=lax.Precision.DEFAULT)
    h1 = h1 + b1_ref[0]
    h2 = lax.dot_general(h1, ws_ref[0], (((1,), (1,)), ((), ())),
                         preferred_element_type=jnp.float32,
                         precision=lax.Precision.DEFAULT)
    h2 = jnp.maximum(h2 + bs_ref[0], 0.0)
    y = lax.dot_general(h2, w2_ref[0], (((1,), (1,)), ((), ())),
                        preferred_element_type=jnp.float32,
                        precision=lax.Precision.DEFAULT)
    ys_ref[...] = (y + b2_ref[0]) * sp_ref[:, 0:1]


def _ffn(te, xs, sp, W1, b1, Ws, bs, W2, b2):
    grid_spec = pltpu.PrefetchScalarGridSpec(
        num_scalar_prefetch=1,
        grid=(NT,),
        in_specs=[
            pl.BlockSpec((TM, EMB), lambda g, te: (g, 0)),
            pl.BlockSpec((TM, 128), lambda g, te: (g, 0)),
            pl.BlockSpec((1, FF, EMB), lambda g, te: (te[g], 0, 0)),
            pl.BlockSpec((1, 1, FF), lambda g, te: (te[g], 0, 0)),
            pl.BlockSpec((1, FF, FF), lambda g, te: (te[g], 0, 0)),
            pl.BlockSpec((1, 1, FF), lambda g, te: (te[g], 0, 0)),
            pl.BlockSpec((1, EMB, FF), lambda g, te: (te[g], 0, 0)),
            pl.BlockSpec((1, 1, EMB), lambda g, te: (te[g], 0, 0)),
        ],
        out_specs=pl.BlockSpec((TM, EMB), lambda g, te: (g, 0)),
    )
    return pl.pallas_call(
        _ffn_body,
        grid_spec=grid_spec,
        out_shape=jax.ShapeDtypeStruct((P, EMB), jnp.float32),
    )(te, xs, sp, W1, b1[:, None], Ws, bs[:, None], W2, b2[:, None])


def _combine_body(ys_hbm, pos0_hbm, pos1_hbm, out_hbm,
                  buf0, buf1, p0_v, p1_v, sem0, sem1):
    wid = lax.axis_index("s") * 2 + lax.axis_index("c")
    base = wid * TOKW
    pltpu.sync_copy(pos0_hbm.at[pl.ds(base, TOKW)], p0_v)
    pltpu.sync_copy(pos1_hbm.at[pl.ds(base, TOKW)], p1_v)
    c0 = pltpu.async_copy(ys_hbm.at[p0_v], buf0, sem0)
    c1 = pltpu.async_copy(ys_hbm.at[p1_v], buf1, sem1)
    c0.wait()
    c1.wait()

    def tok_body(i, _):
        def col_body(j, _):
            sl = pl.ds(j * 16, 16)
            buf0[i, sl] = buf0[i, sl] + buf1[i, sl]
            return 0

        return lax.fori_loop(0, EMB // 16, col_body, 0, unroll=8)

    lax.fori_loop(0, TOKW, tok_body, 0)
    pltpu.sync_copy(buf0, out_hbm.at[pl.ds(base, TOKW)])


def _combine(ys, pos0, pos1):
    mesh = plsc.VectorSubcoreMesh(core_axis_name="c", subcore_axis_name="s")
    fn = functools.partial(
        pl.kernel,
        mesh=mesh,
        out_type=jax.ShapeDtypeStruct((S, EMB), jnp.float32),
        scratch_types=[
            pltpu.VMEM((TOKW, EMB), jnp.float32),
            pltpu.VMEM((TOKW, EMB), jnp.float32),
            pltpu.VMEM((TOKW,), jnp.int32),
            pltpu.VMEM((TOKW,), jnp.int32),
            pltpu.SemaphoreType.DMA,
            pltpu.SemaphoreType.DMA,
        ],
    )(_combine_body)
    return fn(ys, pos0, pos1)


def kernel(x, rW, rb, W1, b1, Ws, bs, W2, b2):
    xf = x.reshape(S, EMB)
    rwt = jnp.zeros((EMB, EP), jnp.float32).at[:, :NE].set(rW.T)
    rbp = jnp.zeros((1, EP), jnp.float32).at[0, :NE].set(rb)

    meta, counts_pad, q0r, q1r = _router(xf, rwt, rbp)

    counts = counts_pad[:, 0, :NE].astype(jnp.int32)
    total = jnp.sum(counts, axis=0)
    ntiles_e = (total + TM - 1) // TM
    off = jnp.concatenate(
        [jnp.zeros((1,), jnp.int32), jnp.cumsum(ntiles_e * TM)[:-1]])
    base = jnp.concatenate(
        [jnp.zeros((1, NE), jnp.int32), jnp.cumsum(counts, axis=0)[:-1]], axis=0)

    e01 = meta[:, 2:4].astype(jnp.int32)
    r01 = meta[:, 4:6].astype(jnp.int32)
    tile_of = (jnp.arange(S, dtype=jnp.int32) // TS)[:, None]
    pos = off[e01] + base[tile_of, e01] + r01
    pos0 = pos[:, 0]
    pos1 = pos[:, 1]

    tile_starts = jnp.cumsum(ntiles_e)
    g = jnp.arange(NT, dtype=jnp.int32)
    te = jnp.minimum(
        jnp.sum((g[:, None] >= tile_starts[None, :]).astype(jnp.int32), axis=1),
        NE - 1).astype(jnp.int32)

    xs, sp = _dispatch(xf, pos0, pos1, q0r, q1r)
    ys = _ffn(te, xs, sp, W1, b1, Ws, bs, W2, b2)
    out = _combine(ys, pos0, pos1)
    return out.reshape(1, S, EMB)
